# baseline (speedup 1.0000x reference)
import functools

import jax
import jax.numpy as jnp
from jax import lax
from jax.experimental import pallas as pl
from jax.experimental.pallas import tpu as pltpu
from jax.experimental.pallas import tpu_sc as plsc

EMBED_DIM = 128
MAX_TIMESTEPS = 1000
VPAD = 1024
BATCH = 16384

_info = plsc.get_sparse_core_info()
_NC, _NS = _info.num_cores, _info.num_subcores
_NW = _NC * _NS
_B_PER_W = BATCH // _NW
_ROWS_PER_TILE = VPAD // _NS

_mesh = plsc.VectorSubcoreMesh(core_axis_name="c", subcore_axis_name="s")

_SIZES = (64, 128, 128, 128, 64)
_OFFS = (0, 64, 192, 320, 448)
_NCHUNK = len(_SIZES)


@functools.partial(
    pl.kernel,
    mesh=_mesh,
    out_type=jax.ShapeDtypeStruct((BATCH, EMBED_DIM), jnp.float32),
    scratch_types=[
        pltpu.VMEM_SHARED((VPAD, EMBED_DIM), jnp.float32),
        pltpu.VMEM((_B_PER_W,), jnp.int32),
        [pltpu.VMEM((s, EMBED_DIM), jnp.float32) for s in _SIZES],
        pltpu.SemaphoreType.DMA,
        pltpu.SemaphoreType.DMA,
        pltpu.SemaphoreType.DMA,
        pltpu.SemaphoreType.DMA,
        pltpu.SemaphoreType.DMA,
    ],
)
def _gather_kernel(
    ts_hbm, pe_hbm, out_hbm, pe_sh, idx_v, bufs, gsem, wsem, ssem, isem, hsem
):
    cid = lax.axis_index("c")
    sid = lax.axis_index("s")
    wid = sid * _NC + cid
    base = wid * _B_PER_W

    idx0 = pltpu.async_copy(
        ts_hbm.at[pl.ds(base, _SIZES[0])], idx_v.at[pl.ds(0, _SIZES[0])], isem
    )
    idx_rest = [
        pltpu.async_copy(
            ts_hbm.at[pl.ds(base + _OFFS[i], _SIZES[i])],
            idx_v.at[pl.ds(_OFFS[i], _SIZES[i])],
            wsem,
        )
        for i in range(1, _NCHUNK)
    ]
    stage = pltpu.async_copy(
        pe_hbm.at[pl.ds(sid * _ROWS_PER_TILE, _ROWS_PER_TILE)],
        pe_sh.at[pl.ds(sid * _ROWS_PER_TILE, _ROWS_PER_TILE)],
        ssem,
    )
    idx0.wait()
    gathers = [
        pltpu.async_copy(
            pe_hbm.at[idx_v.at[pl.ds(0, _SIZES[0])]], bufs[0], hsem
        )
    ]
    for c in idx_rest:
        c.wait()
    stage.wait()
    plsc.subcore_barrier()
    gathers += [
        pltpu.async_copy(
            pe_sh.at[idx_v.at[pl.ds(_OFFS[i], _SIZES[i])]], bufs[i], gsem
        )
        for i in range(1, _NCHUNK)
    ]
    writes = []
    for i in range(_NCHUNK):
        gathers[i].wait()
        writes.append(
            pltpu.async_copy(
                bufs[i], out_hbm.at[pl.ds(base + _OFFS[i], _SIZES[i])], wsem
            )
        )
    for w in writes:
        w.wait()


def kernel(timesteps, pe):
    pe_padded = jnp.zeros((VPAD, EMBED_DIM), jnp.float32).at[:MAX_TIMESTEPS].set(pe)
    return _gather_kernel(timesteps.astype(jnp.int32), pe_padded)
